# baseline (speedup 1.0000x reference)
import jax
import jax.numpy as jnp
import numpy as np
from jax.experimental import pallas as pl

N_NODES = 10000
NODE_TYPE_SIZE = 9


def _mlp(p, x, layer_norm=True):
    h = jax.nn.relu(x @ p['W0'] + p['b0'])
    h = jax.nn.relu(h @ p['W1'] + p['b1'])
    h = h @ p['W2'] + p['b2']
    if layer_norm:
        m = jnp.mean(h, axis=-1, keepdims=True)
        v = jnp.mean((h - m) ** 2, axis=-1, keepdims=True)
        h = (h - m) / jnp.sqrt(v + 1e-5) * p['ln_g'] + p['ln_b']
    return h


def _normalize(data, eps=1e-8):
    count = jnp.maximum(jnp.asarray(data.shape[0], jnp.float32), 1.0)
    mean = jnp.sum(data, axis=0) / count
    var = jnp.sum(data ** 2, axis=0) / count - mean ** 2
    std = jnp.maximum(jnp.sqrt(jnp.maximum(var, 0.0)), eps)
    return (data - mean) / std


def _normalize_masked(data, mask, eps=1e-8):
    m = mask.astype(data.dtype)[:, None]
    count = jnp.maximum(jnp.sum(m), 1.0)
    mean = jnp.sum(data * m, axis=0) / count
    var = jnp.sum((data ** 2) * m, axis=0) / count - mean ** 2
    std = jnp.maximum(jnp.sqrt(jnp.maximum(var, 0.0)), eps)
    return (data - mean) / std


def _id_kernel(x_ref, o_ref):
    o_ref[...] = x_ref[...]


def kernel(world_pos, prev_world_pos, mesh_pos, node_type, cells, params):
    wp = world_pos[0]; pwp = prev_world_pos[0]; mp = mesh_pos[0]
    nt = node_type[0]; fc = cells[0]
    n = wp.shape[0]
    velocity = wp - pwp
    one_hot = jax.nn.one_hot(nt[:, 0], NODE_TYPE_SIZE, dtype=jnp.float32)
    node_features = jnp.concatenate([velocity, one_hot], axis=-1)
    edges = jnp.concatenate([fc[:, 0:2], fc[:, 1:3],
                             jnp.stack([fc[:, 2], fc[:, 0]], axis=1)], axis=0)
    r = jnp.min(edges, axis=1)
    s = jnp.max(edges, axis=1)
    packed = s * n + r
    E = packed.shape[0]
    table = jnp.zeros((n * n,), jnp.int32)
    table = table.at[packed].set(jnp.arange(E, dtype=jnp.int32), mode='drop')
    g = table[packed]
    rep = (g == jnp.arange(E, dtype=jnp.int32))
    senders = jnp.concatenate([s, r], axis=0)
    receivers = jnp.concatenate([r, s], axis=0)
    edge_mask = jnp.concatenate([rep, rep], axis=0)
    rel_wp = jnp.take(wp, senders, axis=0) - jnp.take(wp, receivers, axis=0)
    rel_mp = jnp.take(mp, senders, axis=0) - jnp.take(mp, receivers, axis=0)
    edge_features = jnp.concatenate([
        rel_wp, jnp.linalg.norm(rel_wp, axis=-1, keepdims=True),
        rel_mp, jnp.linalg.norm(rel_mp, axis=-1, keepdims=True)], axis=-1)
    nf = _normalize(node_features)
    ef = _normalize_masked(edge_features, edge_mask)
    node_lat = _mlp(params['node_enc'], nf)
    edge_lat = _mlp(params['edge_enc'], ef)
    mcol = edge_mask.astype(edge_lat.dtype)[:, None]
    for blk in params['proc']:
        edge_in = jnp.concatenate([jnp.take(node_lat, senders, axis=0),
                                   jnp.take(node_lat, receivers, axis=0),
                                   edge_lat], axis=-1)
        edge_lat = edge_lat + _mlp(blk['edge'], edge_in)
        agg = jax.ops.segment_sum(edge_lat * mcol, receivers, num_segments=n)
        node_lat = node_lat + _mlp(blk['node'], jnp.concatenate([node_lat, agg], axis=-1))
    out = _mlp(params['decoder'], node_lat, layer_norm=False)
    out = pl.pallas_call(
        _id_kernel,
        out_shape=jax.ShapeDtypeStruct(out.shape, out.dtype),
    )(out)
    return out


# speedup vs baseline: 1.9695x; 1.9695x over previous
import functools

import jax
import jax.numpy as jnp
from jax import lax
from jax.experimental import pallas as pl
from jax.experimental.pallas import tpu as pltpu
from jax.experimental.pallas import tpu_sc as plsc

N = 10000
NPAD = 10240
L = 128
E3 = 30000
E3P = 32768
EP = 2 * E3P
NBLK = 512
NTS = 9
TBL = 100_008_192

_HI = jax.lax.Precision.HIGHEST


def _mm(a, b):
    return jax.lax.dot_general(a, b, (((1,), (0,)), ((), ())),
                               precision=_HI,
                               preferred_element_type=jnp.float32)


def _relu(x):
    return jnp.maximum(x, 0.0)


def _sc_mesh():
    return plsc.VectorSubcoreMesh(core_axis_name="c", subcore_axis_name="s")


def _sc_repmask(packed_r, ids_r):
    @functools.partial(
        pl.kernel, mesh=_sc_mesh(),
        out_type=jax.ShapeDtypeStruct((TBL,), jnp.int32),
        scratch_types=[pltpu.VMEM((8, 128), jnp.int32),
                       pltpu.VMEM((8, 128), jnp.int32),
                       pltpu.SemaphoreType.DMA],
    )
    def scat(packed_hbm, ids_hbm, table_hbm, pk_v, id_v, sem):
        wid = lax.axis_index("s") * 2 + lax.axis_index("c")
        pltpu.sync_copy(packed_hbm.at[wid], pk_v)
        pltpu.sync_copy(ids_hbm.at[wid], id_v)
        for j in range(8):
            pltpu.async_copy(id_v.at[j], table_hbm.at[pk_v.at[j]], sem).wait()

    table = scat(packed_r, ids_r)

    @functools.partial(
        pl.kernel, mesh=_sc_mesh(),
        out_type=jax.ShapeDtypeStruct((32, 8, 128), jnp.int32),
        scratch_types=[pltpu.VMEM((8, 128), jnp.int32),
                       pltpu.VMEM((8, 128), jnp.int32),
                       pltpu.SemaphoreType.DMA],
    )
    def gath(packed_hbm, table_hbm, g_hbm, pk_v, g_v, sem):
        wid = lax.axis_index("s") * 2 + lax.axis_index("c")
        pltpu.sync_copy(packed_hbm.at[wid], pk_v)
        for j in range(8):
            pltpu.async_copy(table_hbm.at[pk_v.at[j]], g_v.at[j], sem).wait()
        pltpu.sync_copy(g_v, g_hbm.at[wid])

    return gath(packed_r, table)


def _sc_gather(table, idx_r, d):
    @functools.partial(
        pl.kernel, mesh=_sc_mesh(),
        out_type=jax.ShapeDtypeStruct((EP, d), jnp.float32),
        scratch_types=[pltpu.VMEM((16, 128), jnp.int32),
                       pltpu.VMEM((128, d), jnp.float32),
                       pltpu.SemaphoreType.DMA],
    )
    def gath(table_hbm, idx_hbm, out_hbm, idx_v, rows_v, sem):
        wid = lax.axis_index("s") * 2 + lax.axis_index("c")
        pltpu.sync_copy(idx_hbm.at[wid], idx_v)
        base = wid * 2048

        def body(j, carry):
            pltpu.async_copy(table_hbm.at[idx_v.at[j]], rows_v, sem).wait()
            pltpu.sync_copy(rows_v, out_hbm.at[pl.ds(base + j * 128, 128)])
            return carry

        lax.fori_loop(0, 16, body, 0)

    return gath(table, idx_r)


def _sc_scatter_add(vals, sidx_r):
    @functools.partial(
        pl.kernel, mesh=_sc_mesh(),
        out_type=jax.ShapeDtypeStruct((2, NPAD, L), jnp.float32),
        scratch_types=[pltpu.VMEM((16, 128), jnp.int32),
                       pltpu.VMEM((128, L), jnp.float32),
                       pltpu.VMEM_SHARED((NPAD, L), jnp.float32),
                       pltpu.SemaphoreType.DMA],
    )
    def scat(vals_hbm, sidx_hbm, out_hbm, idx_v, rows_v, acc_sh, sem):
        cid = lax.axis_index("c")
        sid = lax.axis_index("s")
        wid = sid * 2 + cid

        zero16 = jnp.zeros((16,), jnp.float32)

        def zb(i, carry):
            for k in range(8):
                rows_v[i, pl.ds(k * 16, 16)] = zero16
            return carry

        lax.fori_loop(0, 128, zb, 0)
        for k in range(5):
            pltpu.sync_copy(rows_v, acc_sh.at[pl.ds(sid * 640 + k * 128, 128)])
        plsc.subcore_barrier()

        pltpu.sync_copy(sidx_hbm.at[wid], idx_v)
        base = wid * 2048

        def body(j, carry):
            pltpu.sync_copy(vals_hbm.at[pl.ds(base + j * 128, 128)], rows_v)
            pltpu.sync_copy(rows_v, acc_sh.at[idx_v.at[j]], add=True)
            return carry

        lax.fori_loop(0, 16, body, 0)
        plsc.subcore_barrier()
        pltpu.sync_copy(acc_sh.at[pl.ds(sid * 640, 640)],
                        out_hbm.at[cid, pl.ds(sid * 640, 640)])

    return scat(vals, sidx_r)


def _node_stats_body(x_ref, o_ref):
    j = pl.program_id(0)
    x = x_ref[...]
    s1 = jnp.sum(x, axis=0, keepdims=True)
    s2 = jnp.sum(x * x, axis=0, keepdims=True)

    @pl.when(j == 0)
    def _():
        o_ref[...] = jnp.zeros((8, 16), jnp.float32)

    o_ref[0:1, 0:12] = o_ref[0:1, 0:12] + s1
    o_ref[1:2, 0:12] = o_ref[1:2, 0:12] + s2


def _tc_node_stats(nf_raw):
    return pl.pallas_call(
        _node_stats_body,
        grid=(5,),
        in_specs=[pl.BlockSpec((2048, 12), lambda j: (j, 0))],
        out_specs=pl.BlockSpec((8, 16), lambda j: (0, 0)),
        out_shape=jax.ShapeDtypeStruct((8, 16), jnp.float32),
    )(nf_raw)


def _enc_node_body(nf_ref, st_ref, w0, b0, w1, b1, w2, b2, g, b, o_ref):
    cnt = jnp.float32(N)
    mean = st_ref[0:1, 0:12] / cnt
    var = st_ref[1:2, 0:12] / cnt - mean * mean
    std = jnp.maximum(jnp.sqrt(jnp.maximum(var, 0.0)), 1e-8)
    x = (nf_ref[...] - mean) / std
    h = _relu(_mm(x, w0[...]) + b0[...])
    h = _relu(_mm(h, w1[...]) + b1[...])
    h = _mm(h, w2[...]) + b2[...]
    m = jnp.mean(h, axis=1, keepdims=True)
    v = jnp.mean((h - m) ** 2, axis=1, keepdims=True)
    o_ref[...] = (h - m) / jnp.sqrt(v + 1e-5) * g[...] + b[...]


def _tc_encode_node(nf_raw, stats, p):
    nb = NPAD // NBLK
    return pl.pallas_call(
        _enc_node_body,
        grid=(nb,),
        in_specs=[pl.BlockSpec((NBLK, 12), lambda j: (j, 0)),
                  pl.BlockSpec((8, 16), lambda j: (0, 0))] +
                 [pl.BlockSpec(w.shape, lambda j: (0,) * w.ndim)
                  for w in (p['W0'], p['b0'], p['W1'], p['b1'], p['W2'],
                            p['b2'], p['ln_g'], p['ln_b'])],
        out_specs=pl.BlockSpec((NBLK, L), lambda j: (j, 0)),
        out_shape=jax.ShapeDtypeStruct((NPAD, L), jnp.float32),
    )(nf_raw, stats, p['W0'], p['b0'], p['W1'], p['b1'], p['W2'], p['b2'],
      p['ln_g'], p['ln_b'])


def _edge_feat(a, b):
    relw = a[:, 0:3] - b[:, 0:3]
    nw = jnp.sqrt(jnp.sum(relw * relw, axis=1, keepdims=True))
    relm = a[:, 3:5] - b[:, 3:5]
    nm = jnp.sqrt(jnp.sum(relm * relm, axis=1, keepdims=True))
    return jnp.concatenate([relw, nw, relm, nm], axis=1)


def _edge_stats_body(a_ref, b_ref, m_ref, o_ref):
    j = pl.program_id(0)
    f = _edge_feat(a_ref[...], b_ref[...])
    m = m_ref[...]
    s1 = jnp.sum(f * m, axis=0, keepdims=True)
    s2 = jnp.sum(f * f * m, axis=0, keepdims=True)
    c = jnp.sum(m).reshape(1, 1)

    @pl.when(j == 0)
    def _():
        o_ref[...] = jnp.zeros((8, 16), jnp.float32)

    o_ref[0:1, 0:7] = o_ref[0:1, 0:7] + s1
    o_ref[1:2, 0:7] = o_ref[1:2, 0:7] + s2
    o_ref[2:3, 0:1] = o_ref[2:3, 0:1] + c


def _tc_edge_stats(g8, rep2f):
    return pl.pallas_call(
        _edge_stats_body,
        grid=(16,),
        in_specs=[pl.BlockSpec((2048, 16), lambda j: (j, 0)),
                  pl.BlockSpec((2048, 16), lambda j: (16 + j, 0)),
                  pl.BlockSpec((2048, 1), lambda j: (j, 0))],
        out_specs=pl.BlockSpec((8, 16), lambda j: (0, 0)),
        out_shape=jax.ShapeDtypeStruct((8, 16), jnp.float32),
    )(g8, g8, rep2f)


def _enc_edge_body(ga_ref, gb_ref, st_ref, w0, b0, w1, b1, w2, b2, g, b,
                   o_ref):
    f = _edge_feat(ga_ref[...], gb_ref[...])
    c = jnp.maximum(st_ref[2, 0], 0.5)
    s1 = st_ref[0:1, 0:7] / c
    s2 = st_ref[1:2, 0:7] / c
    mean = jnp.concatenate([jnp.zeros_like(s1[:, 0:3]), s1[:, 3:4],
                            jnp.zeros_like(s1[:, 4:6]), s1[:, 6:7]], axis=1)
    var = s2 - mean * mean
    std = jnp.maximum(jnp.sqrt(jnp.maximum(var, 0.0)), 1e-8)
    x = (f - mean) / std
    h = _relu(_mm(x, w0[...]) + b0[...])
    h = _relu(_mm(h, w1[...]) + b1[...])
    h = _mm(h, w2[...]) + b2[...]
    m = jnp.mean(h, axis=1, keepdims=True)
    v = jnp.mean((h - m) ** 2, axis=1, keepdims=True)
    o_ref[...] = (h - m) / jnp.sqrt(v + 1e-5) * g[...] + b[...]


def _tc_encode_edge(g8, stats, p):
    nb = EP // NBLK
    half = nb // 2
    return pl.pallas_call(
        _enc_edge_body,
        grid=(nb,),
        in_specs=[pl.BlockSpec((NBLK, 16), lambda j: (j, 0)),
                  pl.BlockSpec((NBLK, 16), lambda j: ((j + half) % nb, 0)),
                  pl.BlockSpec((8, 16), lambda j: (0, 0))] +
                 [pl.BlockSpec(w.shape, lambda j: (0,) * w.ndim)
                  for w in (p['W0'], p['b0'], p['W1'], p['b1'], p['W2'],
                            p['b2'], p['ln_g'], p['ln_b'])],
        out_specs=pl.BlockSpec((NBLK, L), lambda j: (j, 0)),
        out_shape=jax.ShapeDtypeStruct((EP, L), jnp.float32),
    )(g8, g8, stats, p['W0'], p['b0'], p['W1'], p['b1'], p['W2'], p['b2'],
      p['ln_g'], p['ln_b'])


def _edge_step_body(a_ref, b_ref, e_ref, w0, b0, w1, b1, w2, b2, g, b,
                    o_ref):
    e = e_ref[...]
    h = _relu(_mm(a_ref[...], w0[0:L, :]) + _mm(b_ref[...], w0[L:2 * L, :]) +
              _mm(e, w0[2 * L:3 * L, :]) + b0[...])
    h = _relu(_mm(h, w1[...]) + b1[...])
    h = _mm(h, w2[...]) + b2[...]
    m = jnp.mean(h, axis=1, keepdims=True)
    v = jnp.mean((h - m) ** 2, axis=1, keepdims=True)
    h = (h - m) / jnp.sqrt(v + 1e-5) * g[...] + b[...]
    o_ref[...] = e + h


def _tc_edge_step(nlr, el, p):
    nb = EP // NBLK
    half = nb // 2
    return pl.pallas_call(
        _edge_step_body,
        grid=(nb,),
        in_specs=[pl.BlockSpec((NBLK, L), lambda j: (j, 0)),
                  pl.BlockSpec((NBLK, L), lambda j: ((j + half) % nb, 0)),
                  pl.BlockSpec((NBLK, L), lambda j: (j, 0))] +
                 [pl.BlockSpec(w.shape, lambda j: (0,) * w.ndim)
                  for w in (p['W0'], p['b0'], p['W1'], p['b1'], p['W2'],
                            p['b2'], p['ln_g'], p['ln_b'])],
        out_specs=pl.BlockSpec((NBLK, L), lambda j: (j, 0)),
        out_shape=jax.ShapeDtypeStruct((EP, L), jnp.float32),
    )(nlr, nlr, el, p['W0'], p['b0'], p['W1'], p['b1'], p['W2'], p['b2'],
      p['ln_g'], p['ln_b'])


def _node_step_body(nl_ref, a0_ref, a1_ref, w0, b0, w1, b1, w2, b2, g, b,
                    o_ref):
    nl = nl_ref[...]
    agg = a0_ref[0] + a1_ref[0]
    h = _relu(_mm(nl, w0[0:L, :]) + _mm(agg, w0[L:2 * L, :]) + b0[...])
    h = _relu(_mm(h, w1[...]) + b1[...])
    h = _mm(h, w2[...]) + b2[...]
    m = jnp.mean(h, axis=1, keepdims=True)
    v = jnp.mean((h - m) ** 2, axis=1, keepdims=True)
    h = (h - m) / jnp.sqrt(v + 1e-5) * g[...] + b[...]
    o_ref[...] = nl + h


def _tc_node_step(nl, agg, p):
    nb = NPAD // NBLK
    return pl.pallas_call(
        _node_step_body,
        grid=(nb,),
        in_specs=[pl.BlockSpec((NBLK, L), lambda j: (j, 0)),
                  pl.BlockSpec((1, NBLK, L), lambda j: (0, j, 0)),
                  pl.BlockSpec((1, NBLK, L), lambda j: (1, j, 0))] +
                 [pl.BlockSpec(w.shape, lambda j: (0,) * w.ndim)
                  for w in (p['W0'], p['b0'], p['W1'], p['b1'], p['W2'],
                            p['b2'], p['ln_g'], p['ln_b'])],
        out_specs=pl.BlockSpec((NBLK, L), lambda j: (j, 0)),
        out_shape=jax.ShapeDtypeStruct((NPAD, L), jnp.float32),
    )(nl, agg, agg, p['W0'], p['b0'], p['W1'], p['b1'], p['W2'], p['b2'],
      p['ln_g'], p['ln_b'])


def _dec_body(nl_ref, w0, b0, w1, b1, w2, b2, o_ref):
    h = _relu(_mm(nl_ref[...], w0[...]) + b0[...])
    h = _relu(_mm(h, w1[...]) + b1[...])
    o_ref[...] = _mm(h, w2[...]) + b2[...]


def _tc_decode(nl, p):
    nb = NPAD // NBLK
    return pl.pallas_call(
        _dec_body,
        grid=(nb,),
        in_specs=[pl.BlockSpec((NBLK, L), lambda j: (j, 0))] +
                 [pl.BlockSpec(w.shape, lambda j: (0,) * w.ndim)
                  for w in (p['W0'], p['b0'], p['W1'], p['b1'], p['W2'],
                            p['b2'])],
        out_specs=pl.BlockSpec((NBLK, 3), lambda j: (j, 0)),
        out_shape=jax.ShapeDtypeStruct((NPAD, 3), jnp.float32),
    )(nl, p['W0'], p['b0'], p['W1'], p['b1'], p['W2'], p['b2'])


def _prep(p):
    q = dict(p)
    for k in ('b0', 'b1', 'b2', 'ln_g', 'ln_b'):
        if k in q:
            q[k] = q[k].reshape(1, -1)
    return q


def kernel(world_pos, prev_world_pos, mesh_pos, node_type, cells, params):
    wp = world_pos[0]
    pwp = prev_world_pos[0]
    mp = mesh_pos[0]
    nt = node_type[0]
    fc = cells[0]

    edges = jnp.concatenate([fc[:, 0:2], fc[:, 1:3],
                             jnp.stack([fc[:, 2], fc[:, 0]], axis=1)], axis=0)
    r3 = jnp.min(edges, axis=1).astype(jnp.int32)
    s3 = jnp.max(edges, axis=1).astype(jnp.int32)
    padn = E3P - E3
    pad_rows = (jnp.arange(padn, dtype=jnp.int32) * 7919) % N
    s_pad = jnp.concatenate([s3, pad_rows])
    r_pad = jnp.concatenate([r3, pad_rows])
    packed = s3 * N + r3
    packed_pad = jnp.concatenate(
        [packed, 100_000_000 + jnp.arange(padn, dtype=jnp.int32)])
    ids3 = jnp.arange(E3P, dtype=jnp.int32)

    g = _sc_repmask(packed_pad.reshape(32, 8, 128),
                    ids3.reshape(32, 8, 128)).reshape(-1)
    rep3 = (g == ids3) & (ids3 < E3)
    rep2f = jnp.concatenate([rep3, rep3]).astype(jnp.float32).reshape(EP, 1)

    idx2 = jnp.concatenate([s_pad, r_pad])
    idx2_r = idx2.reshape(32, 16, 128)
    rcv2 = jnp.concatenate([r_pad, s_pad])
    ids2 = jnp.arange(EP, dtype=jnp.int32)
    scat_idx = jnp.where(jnp.concatenate([rep3, rep3]), rcv2,
                         N + (ids2 % (NPAD - N)))
    scat_r = scat_idx.reshape(32, 16, 128)

    geo = jnp.zeros((NPAD, 128), jnp.float32)
    geo = geo.at[:N, 0:3].set(wp).at[:N, 3:5].set(mp)
    g8 = _sc_gather(geo, idx2_r, 128)[:, 0:16]
    stats = _tc_edge_stats(g8, rep2f)

    one_hot = jax.nn.one_hot(nt[:, 0], NTS, dtype=jnp.float32)
    nf_raw = jnp.zeros((NPAD, 12), jnp.float32)
    nf_raw = nf_raw.at[:N].set(
        jnp.concatenate([wp - pwp, one_hot], axis=-1))
    nl = _tc_encode_node(nf_raw, _tc_node_stats(nf_raw),
                         _prep(params['node_enc']))
    el = _tc_encode_edge(g8, stats, _prep(params['edge_enc']))

    for blk in params['proc']:
        nlr = _sc_gather(nl, idx2_r, L)
        el = _tc_edge_step(nlr, el, _prep(blk['edge']))
        agg = _sc_scatter_add(el, scat_r)
        nl = _tc_node_step(nl, agg, _prep(blk['node']))

    out = _tc_decode(nl, _prep(params['decoder']))
    return out[:N]


# speedup vs baseline: 2.2688x; 1.1520x over previous
import functools

import jax
import jax.numpy as jnp
from jax import lax
from jax.experimental import pallas as pl
from jax.experimental.pallas import tpu as pltpu
from jax.experimental.pallas import tpu_sc as plsc

N = 10000
NPAD = 10240
L = 128
E3 = 30000
E3P = 32768
EP = 2 * E3P
NBLK = 2048
NTS = 9
TBL = 100_008_192

def _mm(a, b):
    return jax.lax.dot_general(a, b, (((1,), (0,)), ((), ())),
                               precision=jax.lax.Precision.HIGHEST,
                               preferred_element_type=jnp.float32)


def _relu(x):
    return jnp.maximum(x, 0.0)


def _sc_mesh():
    return plsc.VectorSubcoreMesh(core_axis_name="c", subcore_axis_name="s")


def _sc_repmask(packed_r, ids_r):
    @functools.partial(
        pl.kernel, mesh=_sc_mesh(),
        out_type=jax.ShapeDtypeStruct((TBL,), jnp.int32),
        scratch_types=[pltpu.VMEM((8, 128), jnp.int32),
                       pltpu.VMEM((8, 128), jnp.int32),
                       pltpu.SemaphoreType.DMA],
    )
    def scat(packed_hbm, ids_hbm, table_hbm, pk_v, id_v, sem):
        wid = lax.axis_index("s") * 2 + lax.axis_index("c")
        pltpu.sync_copy(packed_hbm.at[wid], pk_v)
        pltpu.sync_copy(ids_hbm.at[wid], id_v)
        for j in range(8):
            pltpu.async_copy(id_v.at[j], table_hbm.at[pk_v.at[j]], sem).wait()

    table = scat(packed_r, ids_r)

    @functools.partial(
        pl.kernel, mesh=_sc_mesh(),
        out_type=jax.ShapeDtypeStruct((32, 8, 128), jnp.int32),
        scratch_types=[pltpu.VMEM((8, 128), jnp.int32),
                       pltpu.VMEM((8, 128), jnp.int32),
                       pltpu.SemaphoreType.DMA],
    )
    def gath(packed_hbm, table_hbm, g_hbm, pk_v, g_v, sem):
        wid = lax.axis_index("s") * 2 + lax.axis_index("c")
        pltpu.sync_copy(packed_hbm.at[wid], pk_v)
        for j in range(8):
            pltpu.async_copy(table_hbm.at[pk_v.at[j]], g_v.at[j], sem).wait()
        pltpu.sync_copy(g_v, g_hbm.at[wid])

    return gath(packed_r, table)


def _sc_gather(table, idx_r, d):
    @functools.partial(
        pl.kernel, mesh=_sc_mesh(),
        out_type=jax.ShapeDtypeStruct((EP, d), jnp.float32),
        scratch_types=[pltpu.VMEM((16, 128), jnp.int32),
                       pltpu.VMEM((128, d), jnp.float32),
                       pltpu.SemaphoreType.DMA],
    )
    def gath(table_hbm, idx_hbm, out_hbm, idx_v, rows_v, sem):
        wid = lax.axis_index("s") * 2 + lax.axis_index("c")
        pltpu.sync_copy(idx_hbm.at[wid], idx_v)
        base = wid * 2048

        def body(j, carry):
            pltpu.async_copy(table_hbm.at[idx_v.at[j]], rows_v, sem).wait()
            pltpu.sync_copy(rows_v, out_hbm.at[pl.ds(base + j * 128, 128)])
            return carry

        lax.fori_loop(0, 16, body, 0)

    return gath(table, idx_r)


def _sc_scatter_add(vals, sidx_r):
    @functools.partial(
        pl.kernel, mesh=_sc_mesh(),
        out_type=jax.ShapeDtypeStruct((2, NPAD, L), jnp.float32),
        scratch_types=[pltpu.VMEM((16, 128), jnp.int32),
                       pltpu.VMEM((128, L), jnp.float32),
                       pltpu.VMEM_SHARED((NPAD, L), jnp.float32),
                       pltpu.SemaphoreType.DMA],
    )
    def scat(vals_hbm, sidx_hbm, out_hbm, idx_v, rows_v, acc_sh, sem):
        cid = lax.axis_index("c")
        sid = lax.axis_index("s")
        wid = sid * 2 + cid

        zero16 = jnp.zeros((16,), jnp.float32)

        def zb(i, carry):
            for k in range(8):
                rows_v[i, pl.ds(k * 16, 16)] = zero16
            return carry

        lax.fori_loop(0, 128, zb, 0)
        for k in range(5):
            pltpu.sync_copy(rows_v, acc_sh.at[pl.ds(sid * 640 + k * 128, 128)])
        plsc.subcore_barrier()

        pltpu.sync_copy(sidx_hbm.at[wid], idx_v)
        base = wid * 2048

        def body(j, carry):
            pltpu.sync_copy(vals_hbm.at[pl.ds(base + j * 128, 128)], rows_v)
            pltpu.sync_copy(rows_v, acc_sh.at[idx_v.at[j]], add=True)
            return carry

        lax.fori_loop(0, 16, body, 0)
        plsc.subcore_barrier()
        pltpu.sync_copy(acc_sh.at[pl.ds(sid * 640, 640)],
                        out_hbm.at[cid, pl.ds(sid * 640, 640)])

    return scat(vals, sidx_r)


def _node_stats_body(x_ref, o_ref):
    j = pl.program_id(0)
    x = x_ref[...]
    s1 = jnp.sum(x, axis=0, keepdims=True)
    s2 = jnp.sum(x * x, axis=0, keepdims=True)

    @pl.when(j == 0)
    def _():
        o_ref[...] = jnp.zeros((8, 16), jnp.float32)

    o_ref[0:1, 0:12] = o_ref[0:1, 0:12] + s1
    o_ref[1:2, 0:12] = o_ref[1:2, 0:12] + s2


def _tc_node_stats(nf_raw):
    return pl.pallas_call(
        _node_stats_body,
        grid=(5,),
        in_specs=[pl.BlockSpec((2048, 12), lambda j: (j, 0))],
        out_specs=pl.BlockSpec((8, 16), lambda j: (0, 0)),
        out_shape=jax.ShapeDtypeStruct((8, 16), jnp.float32),
    )(nf_raw)


def _enc_node_body(nf_ref, st_ref, w0, b0, w1, b1, w2, b2, g, b, o_ref):
    cnt = jnp.float32(N)
    mean = st_ref[0:1, 0:12] / cnt
    var = st_ref[1:2, 0:12] / cnt - mean * mean
    std = jnp.maximum(jnp.sqrt(jnp.maximum(var, 0.0)), 1e-8)
    x = (nf_ref[...] - mean) / std
    h = _relu(_mm(x, w0[...]) + b0[...])
    h = _relu(_mm(h, w1[...]) + b1[...])
    h = _mm(h, w2[...]) + b2[...]
    m = jnp.mean(h, axis=1, keepdims=True)
    v = jnp.mean((h - m) ** 2, axis=1, keepdims=True)
    o_ref[...] = (h - m) / jnp.sqrt(v + 1e-5) * g[...] + b[...]


def _tc_encode_node(nf_raw, stats, p):
    nb = NPAD // NBLK
    return pl.pallas_call(
        _enc_node_body,
        grid=(nb,),
        in_specs=[pl.BlockSpec((NBLK, 12), lambda j: (j, 0)),
                  pl.BlockSpec((8, 16), lambda j: (0, 0))] +
                 [pl.BlockSpec(w.shape, lambda j: (0,) * w.ndim)
                  for w in (p['W0'], p['b0'], p['W1'], p['b1'], p['W2'],
                            p['b2'], p['ln_g'], p['ln_b'])],
        out_specs=pl.BlockSpec((NBLK, L), lambda j: (j, 0)),
        out_shape=jax.ShapeDtypeStruct((NPAD, L), jnp.float32),
    )(nf_raw, stats, p['W0'], p['b0'], p['W1'], p['b1'], p['W2'], p['b2'],
      p['ln_g'], p['ln_b'])


def _edge_feat(a, b):
    relw = a[:, 0:3] - b[:, 0:3]
    nw = jnp.sqrt(jnp.sum(relw * relw, axis=1, keepdims=True))
    relm = a[:, 3:5] - b[:, 3:5]
    nm = jnp.sqrt(jnp.sum(relm * relm, axis=1, keepdims=True))
    return jnp.concatenate([relw, nw, relm, nm], axis=1)


def _edge_stats_body(a_ref, b_ref, m_ref, o_ref):
    j = pl.program_id(0)
    f = _edge_feat(a_ref[...], b_ref[...])
    m = m_ref[...]
    s1 = jnp.sum(f * m, axis=0, keepdims=True)
    s2 = jnp.sum(f * f * m, axis=0, keepdims=True)
    c = jnp.sum(m).reshape(1, 1)

    @pl.when(j == 0)
    def _():
        o_ref[...] = jnp.zeros((8, 16), jnp.float32)

    o_ref[0:1, 0:7] = o_ref[0:1, 0:7] + s1
    o_ref[1:2, 0:7] = o_ref[1:2, 0:7] + s2
    o_ref[2:3, 0:1] = o_ref[2:3, 0:1] + c


def _tc_edge_stats(g8, rep2f):
    return pl.pallas_call(
        _edge_stats_body,
        grid=(16,),
        in_specs=[pl.BlockSpec((2048, 16), lambda j: (j, 0)),
                  pl.BlockSpec((2048, 16), lambda j: (16 + j, 0)),
                  pl.BlockSpec((2048, 1), lambda j: (j, 0))],
        out_specs=pl.BlockSpec((8, 16), lambda j: (0, 0)),
        out_shape=jax.ShapeDtypeStruct((8, 16), jnp.float32),
    )(g8, g8, rep2f)


def _enc_edge_body(ga_ref, gb_ref, st_ref, w0, b0, w1, b1, w2, b2, g, b,
                   o_ref):
    f = _edge_feat(ga_ref[...], gb_ref[...])
    c = jnp.maximum(st_ref[2, 0], 0.5)
    s1 = st_ref[0:1, 0:7] / c
    s2 = st_ref[1:2, 0:7] / c
    mean = jnp.concatenate([jnp.zeros_like(s1[:, 0:3]), s1[:, 3:4],
                            jnp.zeros_like(s1[:, 4:6]), s1[:, 6:7]], axis=1)
    var = s2 - mean * mean
    std = jnp.maximum(jnp.sqrt(jnp.maximum(var, 0.0)), 1e-8)
    x = (f - mean) / std
    h = _relu(_mm(x, w0[...]) + b0[...])
    h = _relu(_mm(h, w1[...]) + b1[...])
    h = _mm(h, w2[...]) + b2[...]
    m = jnp.mean(h, axis=1, keepdims=True)
    v = jnp.mean((h - m) ** 2, axis=1, keepdims=True)
    o_ref[...] = (h - m) / jnp.sqrt(v + 1e-5) * g[...] + b[...]


def _tc_encode_edge(g8, stats, p):
    nb = EP // NBLK
    half = nb // 2
    return pl.pallas_call(
        _enc_edge_body,
        grid=(nb,),
        in_specs=[pl.BlockSpec((NBLK, 16), lambda j: (j, 0)),
                  pl.BlockSpec((NBLK, 16), lambda j: ((j + half) % nb, 0)),
                  pl.BlockSpec((8, 16), lambda j: (0, 0))] +
                 [pl.BlockSpec(w.shape, lambda j: (0,) * w.ndim)
                  for w in (p['W0'], p['b0'], p['W1'], p['b1'], p['W2'],
                            p['b2'], p['ln_g'], p['ln_b'])],
        out_specs=pl.BlockSpec((NBLK, L), lambda j: (j, 0)),
        out_shape=jax.ShapeDtypeStruct((EP, L), jnp.float32),
    )(g8, g8, stats, p['W0'], p['b0'], p['W1'], p['b1'], p['W2'], p['b2'],
      p['ln_g'], p['ln_b'])


def _edge_step_body(a_ref, b_ref, e_ref, w0, b0, w1, b1, w2, b2, g, b,
                    o_ref):
    e = e_ref[...]
    h = _relu(_mm(a_ref[...], w0[0:L, :]) + _mm(b_ref[...], w0[L:2 * L, :]) +
              _mm(e, w0[2 * L:3 * L, :]) + b0[...])
    h = _relu(_mm(h, w1[...]) + b1[...])
    h = _mm(h, w2[...]) + b2[...]
    m = jnp.mean(h, axis=1, keepdims=True)
    v = jnp.mean((h - m) ** 2, axis=1, keepdims=True)
    h = (h - m) / jnp.sqrt(v + 1e-5) * g[...] + b[...]
    o_ref[...] = e + h


def _tc_edge_step(nlr, el, p):
    nb = EP // NBLK
    half = nb // 2
    return pl.pallas_call(
        _edge_step_body,
        grid=(nb,),
        in_specs=[pl.BlockSpec((NBLK, L), lambda j: (j, 0)),
                  pl.BlockSpec((NBLK, L), lambda j: ((j + half) % nb, 0)),
                  pl.BlockSpec((NBLK, L), lambda j: (j, 0))] +
                 [pl.BlockSpec(w.shape, lambda j: (0,) * w.ndim)
                  for w in (p['W0'], p['b0'], p['W1'], p['b1'], p['W2'],
                            p['b2'], p['ln_g'], p['ln_b'])],
        out_specs=pl.BlockSpec((NBLK, L), lambda j: (j, 0)),
        out_shape=jax.ShapeDtypeStruct((EP, L), jnp.float32),
    )(nlr, nlr, el, p['W0'], p['b0'], p['W1'], p['b1'], p['W2'], p['b2'],
      p['ln_g'], p['ln_b'])


def _node_step_body(nl_ref, a0_ref, a1_ref, w0, b0, w1, b1, w2, b2, g, b,
                    o_ref):
    nl = nl_ref[...]
    agg = a0_ref[0] + a1_ref[0]
    h = _relu(_mm(nl, w0[0:L, :]) + _mm(agg, w0[L:2 * L, :]) + b0[...])
    h = _relu(_mm(h, w1[...]) + b1[...])
    h = _mm(h, w2[...]) + b2[...]
    m = jnp.mean(h, axis=1, keepdims=True)
    v = jnp.mean((h - m) ** 2, axis=1, keepdims=True)
    h = (h - m) / jnp.sqrt(v + 1e-5) * g[...] + b[...]
    o_ref[...] = nl + h


def _tc_node_step(nl, agg, p):
    nb = NPAD // NBLK
    return pl.pallas_call(
        _node_step_body,
        grid=(nb,),
        in_specs=[pl.BlockSpec((NBLK, L), lambda j: (j, 0)),
                  pl.BlockSpec((1, NBLK, L), lambda j: (0, j, 0)),
                  pl.BlockSpec((1, NBLK, L), lambda j: (1, j, 0))] +
                 [pl.BlockSpec(w.shape, lambda j: (0,) * w.ndim)
                  for w in (p['W0'], p['b0'], p['W1'], p['b1'], p['W2'],
                            p['b2'], p['ln_g'], p['ln_b'])],
        out_specs=pl.BlockSpec((NBLK, L), lambda j: (j, 0)),
        out_shape=jax.ShapeDtypeStruct((NPAD, L), jnp.float32),
    )(nl, agg, agg, p['W0'], p['b0'], p['W1'], p['b1'], p['W2'], p['b2'],
      p['ln_g'], p['ln_b'])


def _dec_body(nl_ref, w0, b0, w1, b1, w2, b2, o_ref):
    h = _relu(_mm(nl_ref[...], w0[...]) + b0[...])
    h = _relu(_mm(h, w1[...]) + b1[...])
    o_ref[...] = _mm(h, w2[...]) + b2[...]


def _tc_decode(nl, p):
    nb = NPAD // NBLK
    return pl.pallas_call(
        _dec_body,
        grid=(nb,),
        in_specs=[pl.BlockSpec((NBLK, L), lambda j: (j, 0))] +
                 [pl.BlockSpec(w.shape, lambda j: (0,) * w.ndim)
                  for w in (p['W0'], p['b0'], p['W1'], p['b1'], p['W2'],
                            p['b2'])],
        out_specs=pl.BlockSpec((NBLK, 3), lambda j: (j, 0)),
        out_shape=jax.ShapeDtypeStruct((NPAD, 3), jnp.float32),
    )(nl, p['W0'], p['b0'], p['W1'], p['b1'], p['W2'], p['b2'])


def _prep(p):
    q = dict(p)
    for k in ('b0', 'b1', 'b2', 'ln_g', 'ln_b'):
        if k in q:
            q[k] = q[k].reshape(1, -1)
    return q


def kernel(world_pos, prev_world_pos, mesh_pos, node_type, cells, params):
    wp = world_pos[0]
    pwp = prev_world_pos[0]
    mp = mesh_pos[0]
    nt = node_type[0]
    fc = cells[0]

    edges = jnp.concatenate([fc[:, 0:2], fc[:, 1:3],
                             jnp.stack([fc[:, 2], fc[:, 0]], axis=1)], axis=0)
    r3 = jnp.min(edges, axis=1).astype(jnp.int32)
    s3 = jnp.max(edges, axis=1).astype(jnp.int32)
    padn = E3P - E3
    pad_rows = (jnp.arange(padn, dtype=jnp.int32) * 7919) % N
    s_pad = jnp.concatenate([s3, pad_rows])
    r_pad = jnp.concatenate([r3, pad_rows])
    packed = s3 * N + r3
    packed_pad = jnp.concatenate(
        [packed, 100_000_000 + jnp.arange(padn, dtype=jnp.int32)])
    ids3 = jnp.arange(E3P, dtype=jnp.int32)

    g = _sc_repmask(packed_pad.reshape(32, 8, 128),
                    ids3.reshape(32, 8, 128)).reshape(-1)
    rep3 = (g == ids3) & (ids3 < E3)
    rep2f = jnp.concatenate([rep3, rep3]).astype(jnp.float32).reshape(EP, 1)

    idx2 = jnp.concatenate([s_pad, r_pad])
    idx2_r = idx2.reshape(32, 16, 128)
    rcv2 = jnp.concatenate([r_pad, s_pad])
    ids2 = jnp.arange(EP, dtype=jnp.int32)
    scat_idx = jnp.where(jnp.concatenate([rep3, rep3]), rcv2,
                         N + (ids2 % (NPAD - N)))
    scat_r = scat_idx.reshape(32, 16, 128)

    geo = jnp.zeros((NPAD, 128), jnp.float32)
    geo = geo.at[:N, 0:3].set(wp).at[:N, 3:5].set(mp)
    g8 = _sc_gather(geo, idx2_r, 128)[:, 0:16]
    stats = _tc_edge_stats(g8, rep2f)

    one_hot = jax.nn.one_hot(nt[:, 0], NTS, dtype=jnp.float32)
    nf_raw = jnp.zeros((NPAD, 12), jnp.float32)
    nf_raw = nf_raw.at[:N].set(
        jnp.concatenate([wp - pwp, one_hot], axis=-1))
    nl = _tc_encode_node(nf_raw, _tc_node_stats(nf_raw),
                         _prep(params['node_enc']))
    el = _tc_encode_edge(g8, stats, _prep(params['edge_enc']))

    for blk in params['proc']:
        nlr = _sc_gather(nl, idx2_r, L)
        el = _tc_edge_step(nlr, el, _prep(blk['edge']))
        agg = _sc_scatter_add(el, scat_r)
        nl = _tc_node_step(nl, agg, _prep(blk['node']))

    out = _tc_decode(nl, _prep(params['decoder']))
    return out[:N]


# speedup vs baseline: 2.7417x; 1.2085x over previous
import functools

import jax
import jax.numpy as jnp
from jax import lax
from jax.experimental import pallas as pl
from jax.experimental.pallas import tpu as pltpu
from jax.experimental.pallas import tpu_sc as plsc

N = 10000
NPAD = 10240
L = 128
E3 = 30000
E3P = 32768
EP = 2 * E3P
NBLK = 2048
NTS = 9
TBL = 100_008_192

def _mm(a, b):
    return jax.lax.dot_general(a, b, (((1,), (0,)), ((), ())),
                               precision=jax.lax.Precision.HIGHEST,
                               preferred_element_type=jnp.float32)


def _relu(x):
    return jnp.maximum(x, 0.0)


def _sc_mesh():
    return plsc.VectorSubcoreMesh(core_axis_name="c", subcore_axis_name="s")


def _sc_repmask(packed_r, ids_r):
    @functools.partial(
        pl.kernel, mesh=_sc_mesh(),
        out_type=jax.ShapeDtypeStruct((TBL,), jnp.int32),
        scratch_types=[pltpu.VMEM((8, 128), jnp.int32),
                       pltpu.VMEM((8, 128), jnp.int32),
                       pltpu.SemaphoreType.DMA],
    )
    def scat(packed_hbm, ids_hbm, table_hbm, pk_v, id_v, sem):
        wid = lax.axis_index("s") * 2 + lax.axis_index("c")
        pltpu.sync_copy(packed_hbm.at[wid], pk_v)
        pltpu.sync_copy(ids_hbm.at[wid], id_v)
        for j in range(8):
            pltpu.async_copy(id_v.at[j], table_hbm.at[pk_v.at[j]], sem).wait()

    table = scat(packed_r, ids_r)

    @functools.partial(
        pl.kernel, mesh=_sc_mesh(),
        out_type=jax.ShapeDtypeStruct((32, 8, 128), jnp.int32),
        scratch_types=[pltpu.VMEM((8, 128), jnp.int32),
                       pltpu.VMEM((8, 128), jnp.int32),
                       pltpu.SemaphoreType.DMA],
    )
    def gath(packed_hbm, table_hbm, g_hbm, pk_v, g_v, sem):
        wid = lax.axis_index("s") * 2 + lax.axis_index("c")
        pltpu.sync_copy(packed_hbm.at[wid], pk_v)
        for j in range(8):
            pltpu.async_copy(table_hbm.at[pk_v.at[j]], g_v.at[j], sem).wait()
        pltpu.sync_copy(g_v, g_hbm.at[wid])

    return gath(packed_r, table)


def _sc_gather(table, idx_r, d):
    @functools.partial(
        pl.kernel, mesh=_sc_mesh(),
        out_type=jax.ShapeDtypeStruct((EP, d), jnp.float32),
        scratch_types=[pltpu.VMEM((16, 128), jnp.int32),
                       pltpu.VMEM((128, d), jnp.float32),
                       pltpu.SemaphoreType.DMA],
    )
    def gath(table_hbm, idx_hbm, out_hbm, idx_v, rows_v, sem):
        wid = lax.axis_index("s") * 2 + lax.axis_index("c")
        pltpu.sync_copy(idx_hbm.at[wid], idx_v)
        base = wid * 2048

        def body(j, carry):
            pltpu.async_copy(table_hbm.at[idx_v.at[j]], rows_v, sem).wait()
            pltpu.sync_copy(rows_v, out_hbm.at[pl.ds(base + j * 128, 128)])
            return carry

        lax.fori_loop(0, 16, body, 0)

    return gath(table, idx_r)


def _sc_scatter_add(vals, sidx_r):
    @functools.partial(
        pl.kernel, mesh=_sc_mesh(),
        out_type=jax.ShapeDtypeStruct((2, NPAD, L), jnp.float32),
        scratch_types=[pltpu.VMEM((16, 128), jnp.int32),
                       pltpu.VMEM((128, L), jnp.float32),
                       pltpu.VMEM_SHARED((NPAD, L), jnp.float32),
                       pltpu.SemaphoreType.DMA],
    )
    def scat(vals_hbm, sidx_hbm, out_hbm, idx_v, rows_v, acc_sh, sem):
        cid = lax.axis_index("c")
        sid = lax.axis_index("s")
        wid = sid * 2 + cid

        zero16 = jnp.zeros((16,), jnp.float32)

        def zb(i, carry):
            for k in range(8):
                rows_v[i, pl.ds(k * 16, 16)] = zero16
            return carry

        lax.fori_loop(0, 128, zb, 0)
        for k in range(5):
            pltpu.sync_copy(rows_v, acc_sh.at[pl.ds(sid * 640 + k * 128, 128)])
        plsc.subcore_barrier()

        pltpu.sync_copy(sidx_hbm.at[wid], idx_v)
        base = wid * 2048

        def body(j, carry):
            pltpu.sync_copy(vals_hbm.at[pl.ds(base + j * 128, 128)], rows_v)
            pltpu.sync_copy(rows_v, acc_sh.at[idx_v.at[j]], add=True)
            return carry

        lax.fori_loop(0, 16, body, 0)
        plsc.subcore_barrier()
        pltpu.sync_copy(acc_sh.at[pl.ds(sid * 640, 640)],
                        out_hbm.at[cid, pl.ds(sid * 640, 640)])

    return scat(vals, sidx_r)


def _node_stats_body(x_ref, o_ref):
    j = pl.program_id(0)
    x = x_ref[...]
    s1 = jnp.sum(x, axis=0, keepdims=True)
    s2 = jnp.sum(x * x, axis=0, keepdims=True)

    @pl.when(j == 0)
    def _():
        o_ref[...] = jnp.zeros((8, 16), jnp.float32)

    o_ref[0:1, 0:12] = o_ref[0:1, 0:12] + s1
    o_ref[1:2, 0:12] = o_ref[1:2, 0:12] + s2


def _tc_node_stats(nf_raw):
    return pl.pallas_call(
        _node_stats_body,
        grid=(5,),
        in_specs=[pl.BlockSpec((2048, 12), lambda j: (j, 0))],
        out_specs=pl.BlockSpec((8, 16), lambda j: (0, 0)),
        out_shape=jax.ShapeDtypeStruct((8, 16), jnp.float32),
    )(nf_raw)


def _enc_node_body(nf_ref, st_ref, w0, b0, w1, b1, w2, b2, g, b, o_ref):
    cnt = jnp.float32(N)
    mean = st_ref[0:1, 0:12] / cnt
    var = st_ref[1:2, 0:12] / cnt - mean * mean
    std = jnp.maximum(jnp.sqrt(jnp.maximum(var, 0.0)), 1e-8)
    x = (nf_ref[...] - mean) / std
    h = _relu(_mm(x, w0[...]) + b0[...])
    h = _relu(_mm(h, w1[...]) + b1[...])
    h = _mm(h, w2[...]) + b2[...]
    m = jnp.mean(h, axis=1, keepdims=True)
    v = jnp.mean((h - m) ** 2, axis=1, keepdims=True)
    o_ref[...] = (h - m) / jnp.sqrt(v + 1e-5) * g[...] + b[...]


def _tc_encode_node(nf_raw, stats, p):
    nb = NPAD // NBLK
    return pl.pallas_call(
        _enc_node_body,
        grid=(nb,),
        in_specs=[pl.BlockSpec((NBLK, 12), lambda j: (j, 0)),
                  pl.BlockSpec((8, 16), lambda j: (0, 0))] +
                 [pl.BlockSpec(w.shape, lambda j: (0,) * w.ndim)
                  for w in (p['W0'], p['b0'], p['W1'], p['b1'], p['W2'],
                            p['b2'], p['ln_g'], p['ln_b'])],
        out_specs=pl.BlockSpec((NBLK, L), lambda j: (j, 0)),
        out_shape=jax.ShapeDtypeStruct((NPAD, L), jnp.float32),
    )(nf_raw, stats, p['W0'], p['b0'], p['W1'], p['b1'], p['W2'], p['b2'],
      p['ln_g'], p['ln_b'])


def _edge_feat(a, b):
    relw = a[:, 0:3] - b[:, 0:3]
    nw = jnp.sqrt(jnp.sum(relw * relw, axis=1, keepdims=True))
    relm = a[:, 3:5] - b[:, 3:5]
    nm = jnp.sqrt(jnp.sum(relm * relm, axis=1, keepdims=True))
    return jnp.concatenate([relw, nw, relm, nm], axis=1)


def _edge_stats_body(a_ref, b_ref, m_ref, o_ref):
    j = pl.program_id(0)
    f = _edge_feat(a_ref[...], b_ref[...])
    m = m_ref[...]
    s1 = jnp.sum(f * m, axis=0, keepdims=True)
    s2 = jnp.sum(f * f * m, axis=0, keepdims=True)
    c = jnp.sum(m).reshape(1, 1)

    @pl.when(j == 0)
    def _():
        o_ref[...] = jnp.zeros((8, 16), jnp.float32)

    o_ref[0:1, 0:7] = o_ref[0:1, 0:7] + s1
    o_ref[1:2, 0:7] = o_ref[1:2, 0:7] + s2
    o_ref[2:3, 0:1] = o_ref[2:3, 0:1] + c


def _tc_edge_stats(g8, rep2f):
    return pl.pallas_call(
        _edge_stats_body,
        grid=(16,),
        in_specs=[pl.BlockSpec((2048, 16), lambda j: (j, 0)),
                  pl.BlockSpec((2048, 16), lambda j: (16 + j, 0)),
                  pl.BlockSpec((2048, 1), lambda j: (j, 0))],
        out_specs=pl.BlockSpec((8, 16), lambda j: (0, 0)),
        out_shape=jax.ShapeDtypeStruct((8, 16), jnp.float32),
    )(g8, g8, rep2f)


def _enc_edge_body(ga_ref, gb_ref, st_ref, w0, b0, w1, b1, w2, b2, g, b,
                   o_ref):
    f = _edge_feat(ga_ref[...], gb_ref[...])
    c = jnp.maximum(st_ref[2, 0], 0.5)
    s1 = st_ref[0:1, 0:7] / c
    s2 = st_ref[1:2, 0:7] / c
    mean = jnp.concatenate([jnp.zeros_like(s1[:, 0:3]), s1[:, 3:4],
                            jnp.zeros_like(s1[:, 4:6]), s1[:, 6:7]], axis=1)
    var = s2 - mean * mean
    std = jnp.maximum(jnp.sqrt(jnp.maximum(var, 0.0)), 1e-8)
    x = (f - mean) / std
    h = _relu(_mm(x, w0[...]) + b0[...])
    h = _relu(_mm(h, w1[...]) + b1[...])
    h = _mm(h, w2[...]) + b2[...]
    m = jnp.mean(h, axis=1, keepdims=True)
    v = jnp.mean((h - m) ** 2, axis=1, keepdims=True)
    o_ref[...] = (h - m) / jnp.sqrt(v + 1e-5) * g[...] + b[...]


def _tc_encode_edge(g8, stats, p):
    nb = EP // NBLK
    half = nb // 2
    return pl.pallas_call(
        _enc_edge_body,
        grid=(nb,),
        in_specs=[pl.BlockSpec((NBLK, 16), lambda j: (j, 0)),
                  pl.BlockSpec((NBLK, 16), lambda j: ((j + half) % nb, 0)),
                  pl.BlockSpec((8, 16), lambda j: (0, 0))] +
                 [pl.BlockSpec(w.shape, lambda j: (0,) * w.ndim)
                  for w in (p['W0'], p['b0'], p['W1'], p['b1'], p['W2'],
                            p['b2'], p['ln_g'], p['ln_b'])],
        out_specs=pl.BlockSpec((NBLK, L), lambda j: (j, 0)),
        out_shape=jax.ShapeDtypeStruct((EP, L), jnp.float32),
    )(g8, g8, stats, p['W0'], p['b0'], p['W1'], p['b1'], p['W2'], p['b2'],
      p['ln_g'], p['ln_b'])


def _edge_step_body(a_ref, b_ref, e_ref, w0e, b0, w1, b1, w2, b2, g, b,
                    o_ref):
    e = e_ref[...]
    h = _relu(a_ref[:, 0:L] + b_ref[:, L:2 * L] + _mm(e, w0e[...]) + b0[...])
    h = _relu(_mm(h, w1[...]) + b1[...])
    h = _mm(h, w2[...]) + b2[...]
    m = jnp.mean(h, axis=1, keepdims=True)
    v = jnp.mean((h - m) ** 2, axis=1, keepdims=True)
    h = (h - m) / jnp.sqrt(v + 1e-5) * g[...] + b[...]
    o_ref[...] = e + h


def _tc_edge_step(pqr, el, p):
    nb = EP // NBLK
    half = nb // 2
    w0e = p['W0'][2 * L:3 * L, :]
    return pl.pallas_call(
        _edge_step_body,
        grid=(nb,),
        in_specs=[pl.BlockSpec((NBLK, 2 * L), lambda j: (j, 0)),
                  pl.BlockSpec((NBLK, 2 * L), lambda j: ((j + half) % nb, 0)),
                  pl.BlockSpec((NBLK, L), lambda j: (j, 0))] +
                 [pl.BlockSpec(w.shape, lambda j: (0,) * w.ndim)
                  for w in (w0e, p['b0'], p['W1'], p['b1'], p['W2'],
                            p['b2'], p['ln_g'], p['ln_b'])],
        out_specs=pl.BlockSpec((NBLK, L), lambda j: (j, 0)),
        out_shape=jax.ShapeDtypeStruct((EP, L), jnp.float32),
    )(pqr, pqr, el, w0e, p['b0'], p['W1'], p['b1'], p['W2'], p['b2'],
      p['ln_g'], p['ln_b'])


def _node_step_body(nl_ref, a0_ref, a1_ref, w0, b0, w1, b1, w2, b2, g, b,
                    wpq, o_ref, pq_ref):
    nl = nl_ref[...]
    agg = a0_ref[0] + a1_ref[0]
    h = _relu(_mm(nl, w0[0:L, :]) + _mm(agg, w0[L:2 * L, :]) + b0[...])
    h = _relu(_mm(h, w1[...]) + b1[...])
    h = _mm(h, w2[...]) + b2[...]
    m = jnp.mean(h, axis=1, keepdims=True)
    v = jnp.mean((h - m) ** 2, axis=1, keepdims=True)
    h = (h - m) / jnp.sqrt(v + 1e-5) * g[...] + b[...]
    nlo = nl + h
    o_ref[...] = nlo
    pq_ref[...] = _mm(nlo, wpq[...])


def _tc_node_step(nl, agg, p, wpq):
    nb = NPAD // NBLK
    return pl.pallas_call(
        _node_step_body,
        grid=(nb,),
        in_specs=[pl.BlockSpec((NBLK, L), lambda j: (j, 0)),
                  pl.BlockSpec((1, NBLK, L), lambda j: (0, j, 0)),
                  pl.BlockSpec((1, NBLK, L), lambda j: (1, j, 0))] +
                 [pl.BlockSpec(w.shape, lambda j: (0,) * w.ndim)
                  for w in (p['W0'], p['b0'], p['W1'], p['b1'], p['W2'],
                            p['b2'], p['ln_g'], p['ln_b'], wpq)],
        out_specs=[pl.BlockSpec((NBLK, L), lambda j: (j, 0)),
                   pl.BlockSpec((NBLK, 2 * L), lambda j: (j, 0))],
        out_shape=[jax.ShapeDtypeStruct((NPAD, L), jnp.float32),
                   jax.ShapeDtypeStruct((NPAD, 2 * L), jnp.float32)],
    )(nl, agg, agg, p['W0'], p['b0'], p['W1'], p['b1'], p['W2'], p['b2'],
      p['ln_g'], p['ln_b'], wpq)


def _proj_body(nl_ref, wpq, pq_ref):
    pq_ref[...] = _mm(nl_ref[...], wpq[...])


def _tc_project(nl, wpq):
    nb = NPAD // NBLK
    return pl.pallas_call(
        _proj_body,
        grid=(nb,),
        in_specs=[pl.BlockSpec((NBLK, L), lambda j: (j, 0)),
                  pl.BlockSpec(wpq.shape, lambda j: (0, 0))],
        out_specs=pl.BlockSpec((NBLK, 2 * L), lambda j: (j, 0)),
        out_shape=jax.ShapeDtypeStruct((NPAD, 2 * L), jnp.float32),
    )(nl, wpq)


def _dec_body(nl_ref, w0, b0, w1, b1, w2, b2, o_ref):
    h = _relu(_mm(nl_ref[...], w0[...]) + b0[...])
    h = _relu(_mm(h, w1[...]) + b1[...])
    o_ref[...] = _mm(h, w2[...]) + b2[...]


def _tc_decode(nl, p):
    nb = NPAD // NBLK
    return pl.pallas_call(
        _dec_body,
        grid=(nb,),
        in_specs=[pl.BlockSpec((NBLK, L), lambda j: (j, 0))] +
                 [pl.BlockSpec(w.shape, lambda j: (0,) * w.ndim)
                  for w in (p['W0'], p['b0'], p['W1'], p['b1'], p['W2'],
                            p['b2'])],
        out_specs=pl.BlockSpec((NBLK, 3), lambda j: (j, 0)),
        out_shape=jax.ShapeDtypeStruct((NPAD, 3), jnp.float32),
    )(nl, p['W0'], p['b0'], p['W1'], p['b1'], p['W2'], p['b2'])


def _prep(p):
    q = dict(p)
    for k in ('b0', 'b1', 'b2', 'ln_g', 'ln_b'):
        if k in q:
            q[k] = q[k].reshape(1, -1)
    return q


def kernel(world_pos, prev_world_pos, mesh_pos, node_type, cells, params):
    wp = world_pos[0]
    pwp = prev_world_pos[0]
    mp = mesh_pos[0]
    nt = node_type[0]
    fc = cells[0]

    edges = jnp.concatenate([fc[:, 0:2], fc[:, 1:3],
                             jnp.stack([fc[:, 2], fc[:, 0]], axis=1)], axis=0)
    r3 = jnp.min(edges, axis=1).astype(jnp.int32)
    s3 = jnp.max(edges, axis=1).astype(jnp.int32)
    padn = E3P - E3
    pad_rows = (jnp.arange(padn, dtype=jnp.int32) * 7919) % N
    s_pad = jnp.concatenate([s3, pad_rows])
    r_pad = jnp.concatenate([r3, pad_rows])
    packed = s3 * N + r3
    packed_pad = jnp.concatenate(
        [packed, 100_000_000 + jnp.arange(padn, dtype=jnp.int32)])
    ids3 = jnp.arange(E3P, dtype=jnp.int32)

    g = _sc_repmask(packed_pad.reshape(32, 8, 128),
                    ids3.reshape(32, 8, 128)).reshape(-1)
    rep3 = (g == ids3) & (ids3 < E3)
    rep2f = jnp.concatenate([rep3, rep3]).astype(jnp.float32).reshape(EP, 1)

    idx2 = jnp.concatenate([s_pad, r_pad])
    idx2_r = idx2.reshape(32, 16, 128)
    rcv2 = jnp.concatenate([r_pad, s_pad])
    ids2 = jnp.arange(EP, dtype=jnp.int32)
    scat_idx = jnp.where(jnp.concatenate([rep3, rep3]), rcv2,
                         N + (ids2 % (NPAD - N)))
    scat_r = scat_idx.reshape(32, 16, 128)

    geo = jnp.zeros((NPAD, 128), jnp.float32)
    geo = geo.at[:N, 0:3].set(wp).at[:N, 3:5].set(mp)
    g8 = _sc_gather(geo, idx2_r, 128)[:, 0:16]
    stats = _tc_edge_stats(g8, rep2f)

    one_hot = jax.nn.one_hot(nt[:, 0], NTS, dtype=jnp.float32)
    nf_raw = jnp.zeros((NPAD, 12), jnp.float32)
    nf_raw = nf_raw.at[:N].set(
        jnp.concatenate([wp - pwp, one_hot], axis=-1))
    nl = _tc_encode_node(nf_raw, _tc_node_stats(nf_raw),
                         _prep(params['node_enc']))
    el = _tc_encode_edge(g8, stats, _prep(params['edge_enc']))

    def wpq_of(i):
        if i >= len(params['proc']):
            return jnp.zeros((L, 2 * L), jnp.float32)
        w0 = params['proc'][i]['edge']['W0']
        return jnp.concatenate([w0[0:L, :], w0[L:2 * L, :]], axis=1)

    pq = _tc_project(nl, wpq_of(0))
    for i, blk in enumerate(params['proc']):
        pqr = _sc_gather(pq, idx2_r, 2 * L)
        el = _tc_edge_step(pqr, el, _prep(blk['edge']))
        agg = _sc_scatter_add(el, scat_r)
        nl, pq = _tc_node_step(nl, agg, _prep(blk['node']), wpq_of(i + 1))

    out = _tc_decode(nl, _prep(params['decoder']))
    return out[:N]
